# Initial kernel scaffold; baseline (speedup 1.0000x reference)
import jax
import jax.numpy as jnp
from jax.experimental import pallas as pl


def kernel(token_ids, embeddings):
    raise NotImplementedError("write your pallas kernel here")


# speedup vs baseline: 1.1881x; 1.1881x over previous
import functools

import jax
import jax.numpy as jnp
from jax import lax
from jax.experimental import pallas as pl
from jax.experimental.pallas import tpu as pltpu
from jax.experimental.pallas import tpu_sc as plsc

_CHUNK = 128


@functools.lru_cache(maxsize=None)
def _build(num_rows, dim, batch):
    info = plsc.get_sparse_core_info()
    nc, ns = info.num_cores, info.num_subcores
    nw = nc * ns
    per_w = batch // nw
    steps = per_w // _CHUNK
    assert per_w % _CHUNK == 0 and batch % nw == 0

    mesh = plsc.VectorSubcoreMesh(core_axis_name="c", subcore_axis_name="s")

    @functools.partial(
        pl.kernel,
        mesh=mesh,
        compiler_params=pltpu.CompilerParams(use_tc_tiling_on_sc=False),
        out_type=jax.ShapeDtypeStruct((nw, steps, _CHUNK, dim), jnp.float32),
        scratch_types=[
            pltpu.VMEM((steps, _CHUNK), jnp.int32),
            pltpu.VMEM((_CHUNK, dim), jnp.float32),
            pltpu.SemaphoreType.DMA,
        ],
    )
    def gather_kernel(table, idxs, out, idx_v, rows_v, sem):
        wid = lax.axis_index("s") * nc + lax.axis_index("c")
        pltpu.sync_copy(idxs.at[wid], idx_v)

        def step(j, carry):
            pltpu.async_copy(table.at[idx_v.at[j]], rows_v, sem).wait()
            pltpu.sync_copy(rows_v, out.at[wid, j])
            return carry

        lax.fori_loop(0, steps, step, 0)

    return gather_kernel, nw, steps


def kernel(token_ids, embeddings):
    b, s = token_ids.shape
    v, d = embeddings.shape
    batch = b * s
    gather_kernel, nw, steps = _build(v, d, batch)
    idx = token_ids.reshape(nw, steps, _CHUNK)
    out = gather_kernel(embeddings, idx)
    return out.reshape(b, s, d)


# speedup vs baseline: 1.2285x; 1.0340x over previous
import functools

import jax
import jax.numpy as jnp
from jax import lax
from jax.experimental import pallas as pl
from jax.experimental.pallas import tpu as pltpu
from jax.experimental.pallas import tpu_sc as plsc

_CHUNK = 1024


@functools.lru_cache(maxsize=None)
def _build(num_rows, dim, batch):
    info = plsc.get_sparse_core_info()
    nc, ns = info.num_cores, info.num_subcores
    nw = nc * ns
    per_w = batch // nw
    steps = per_w // _CHUNK
    assert per_w % _CHUNK == 0 and batch % nw == 0

    mesh = plsc.VectorSubcoreMesh(core_axis_name="c", subcore_axis_name="s")

    @functools.partial(
        pl.kernel,
        mesh=mesh,
        compiler_params=pltpu.CompilerParams(use_tc_tiling_on_sc=False),
        out_type=jax.ShapeDtypeStruct((nw, steps, _CHUNK, dim), jnp.float32),
        scratch_types=[
            pltpu.VMEM((steps, _CHUNK), jnp.int32),
            pltpu.VMEM((_CHUNK, dim), jnp.float32),
            pltpu.SemaphoreType.DMA,
        ],
    )
    def gather_kernel(table, idxs, out, idx_v, rows_v, sem):
        wid = lax.axis_index("s") * nc + lax.axis_index("c")
        pltpu.sync_copy(idxs.at[wid], idx_v)

        def step(j, carry):
            pltpu.async_copy(table.at[idx_v.at[j]], rows_v, sem).wait()
            pltpu.sync_copy(rows_v, out.at[wid, j])
            return carry

        lax.fori_loop(0, steps, step, 0)

    return gather_kernel, nw, steps


def kernel(token_ids, embeddings):
    b, s = token_ids.shape
    v, d = embeddings.shape
    batch = b * s
    gather_kernel, nw, steps = _build(v, d, batch)
    idx = token_ids.reshape(nw, steps, _CHUNK)
    out = gather_kernel(embeddings, idx)
    return out.reshape(b, s, d)


# speedup vs baseline: 1.2740x; 1.0371x over previous
import functools

import jax
import jax.numpy as jnp
from jax import lax
from jax.experimental import pallas as pl
from jax.experimental.pallas import tpu as pltpu
from jax.experimental.pallas import tpu_sc as plsc

_CHUNK = 512
_NBUF = 4


@functools.lru_cache(maxsize=None)
def _build(num_rows, dim, batch):
    info = plsc.get_sparse_core_info()
    nc, ns = info.num_cores, info.num_subcores
    nw = nc * ns
    per_w = batch // nw
    steps = per_w // _CHUNK
    assert per_w % _CHUNK == 0 and batch % nw == 0

    mesh = plsc.VectorSubcoreMesh(core_axis_name="c", subcore_axis_name="s")

    @functools.partial(
        pl.kernel,
        mesh=mesh,
        compiler_params=pltpu.CompilerParams(use_tc_tiling_on_sc=False),
        out_type=jax.ShapeDtypeStruct((nw, steps, _CHUNK, dim), jnp.float32),
        scratch_types=[
            pltpu.VMEM((steps, _CHUNK), jnp.int32),
            pltpu.VMEM((_NBUF, _CHUNK, dim), jnp.float32),
            pltpu.SemaphoreType.DMA,
            pltpu.SemaphoreType.DMA,
        ],
    )
    def gather_kernel(table, idxs, out, idx_v, rows_v, gsem, wsem):
        wid = lax.axis_index("s") * nc + lax.axis_index("c")
        pltpu.sync_copy(idxs.at[wid], idx_v)

        gathers = [None] * steps
        writes = [None] * steps

        def start_gather(j):
            return pltpu.async_copy(
                table.at[idx_v.at[j]], rows_v.at[j % _NBUF], gsem
            )

        gathers[0] = start_gather(0)
        for j in range(steps):
            if j + 1 < steps:
                if j + 1 >= _NBUF:
                    writes[j + 1 - _NBUF].wait()
                gathers[j + 1] = start_gather(j + 1)
            gathers[j].wait()
            writes[j] = pltpu.async_copy(
                rows_v.at[j % _NBUF], out.at[wid, j], wsem
            )
        for j in range(max(0, steps - _NBUF), steps):
            writes[j].wait()

    return gather_kernel, nw, steps


def kernel(token_ids, embeddings):
    b, s = token_ids.shape
    v, d = embeddings.shape
    batch = b * s
    gather_kernel, nw, steps = _build(v, d, batch)
    idx = token_ids.reshape(nw, steps, _CHUNK)
    out = gather_kernel(embeddings, idx)
    return out.reshape(b, s, d)


# speedup vs baseline: 1.5016x; 1.1786x over previous
import functools

import jax
import jax.numpy as jnp
from jax import lax
from jax.experimental import pallas as pl
from jax.experimental.pallas import tpu as pltpu
from jax.experimental.pallas import tpu_sc as plsc

_NBUF = 2


@functools.lru_cache(maxsize=None)
def _build(num_rows, dim, b, s):
    info = plsc.get_sparse_core_info()
    nc, ns, nl = info.num_cores, info.num_subcores, info.num_lanes
    nw = nc * ns
    bw = b // nw
    assert b % nw == 0 and bw % nl == 0 and dim % nl == 0

    mesh = plsc.VectorSubcoreMesh(core_axis_name="c", subcore_axis_name="s")

    @functools.partial(
        pl.kernel,
        mesh=mesh,
        compiler_params=pltpu.CompilerParams(
            use_tc_tiling_on_sc=False, needs_layout_passes=False
        ),
        out_type=jax.ShapeDtypeStruct((s, dim, b), jnp.float32),
        scratch_types=[
            pltpu.VMEM((s, bw), jnp.int32),
            pltpu.VMEM((_NBUF, bw, dim), jnp.float32),
            pltpu.VMEM((_NBUF, dim, bw), jnp.float32),
            pltpu.SemaphoreType.DMA,
            pltpu.SemaphoreType.DMA,
        ],
    )
    def gather_kernel(table, idxs, out, idx_v, rows_v, trans_v, gsem, wsem):
        wid = lax.axis_index("s") * nc + lax.axis_index("c")
        base = wid * bw
        pltpu.sync_copy(idxs.at[:, pl.ds(base, bw)], idx_v)

        iota = lax.iota(jnp.int32, nl)

        def start_gather(c):
            return pltpu.async_copy(
                table.at[idx_v.at[c]], rows_v.at[lax.rem(c, _NBUF)], gsem
            )

        start_gather(0)

        def loop_body(c, carry):
            cm = lax.rem(c, _NBUF)

            @pl.when(c + 1 < s)
            def _():
                start_gather(c + 1)

            pltpu.make_async_copy(
                table.at[idx_v.at[c]], rows_v.at[cm], gsem
            ).wait()

            @pl.when(c >= _NBUF)
            def _():
                pltpu.make_async_copy(
                    trans_v.at[cm],
                    out.at[c - _NBUF, :, pl.ds(base, bw)],
                    wsem,
                ).wait()

            rows = rows_v.at[cm]
            trans = trans_v.at[cm]

            def body(g, cc):
                row_ids = g * nl + iota
                for d in range(dim):
                    v = plsc.load_gather(
                        rows, [row_ids, jnp.full((nl,), d, jnp.int32)]
                    )
                    trans[d, pl.ds(g * nl, nl)] = v
                return cc

            lax.fori_loop(0, bw // nl, body, 0)

            pltpu.async_copy(
                trans_v.at[cm], out.at[c, :, pl.ds(base, bw)], wsem
            )
            return carry

        lax.fori_loop(0, s, loop_body, 0)
        for k in range(_NBUF):
            c = s - _NBUF + k
            pltpu.make_async_copy(
                trans_v.at[c % _NBUF], out.at[c, :, pl.ds(base, bw)], wsem
            ).wait()

    return gather_kernel


def kernel(token_ids, embeddings):
    b, s = token_ids.shape
    v, d = embeddings.shape
    gather_kernel = _build(v, d, b, s)
    out_sdb = gather_kernel(embeddings, token_ids.T)
    return jnp.transpose(out_sdb, (2, 0, 1))


# speedup vs baseline: 1.6600x; 1.1055x over previous
import functools

import jax
import jax.numpy as jnp
from jax import lax
from jax.experimental import pallas as pl
from jax.experimental.pallas import tpu as pltpu
from jax.experimental.pallas import tpu_sc as plsc

_NBUF = 2


@functools.lru_cache(maxsize=None)
def _build(num_rows, dim, b, s):
    info = plsc.get_sparse_core_info()
    nc, ns, nl = info.num_cores, info.num_subcores, info.num_lanes
    nw = nc * ns
    bw = b // nw
    assert b % nw == 0 and bw % nl == 0 and dim % nl == 0

    mesh = plsc.VectorSubcoreMesh(core_axis_name="c", subcore_axis_name="s")

    @functools.partial(
        pl.kernel,
        mesh=mesh,
        compiler_params=pltpu.CompilerParams(
            use_tc_tiling_on_sc=False, needs_layout_passes=False
        ),
        out_type=jax.ShapeDtypeStruct((s, dim, b), jnp.float32),
        scratch_types=[
            pltpu.VMEM((s, bw), jnp.int32),
            pltpu.VMEM((_NBUF, bw, dim), jnp.float32),
            pltpu.VMEM((_NBUF, dim, bw), jnp.float32),
            pltpu.SemaphoreType.DMA,
            pltpu.SemaphoreType.DMA,
        ],
    )
    def gather_kernel(table, idxs, out, idx_v, rows_v, trans_v, gsem, wsem):
        wid = lax.axis_index("s") * nc + lax.axis_index("c")
        base = wid * bw
        pltpu.sync_copy(idxs.at[:, pl.ds(base, bw)], idx_v)

        iota = lax.iota(jnp.int32, nl)

        def start_gather(c):
            return pltpu.async_copy(
                table.at[idx_v.at[c]], rows_v.at[lax.rem(c, _NBUF)], gsem
            )

        start_gather(0)

        def loop_body(c, carry):
            cm = lax.rem(c, _NBUF)

            @pl.when(c + 1 < s)
            def _():
                start_gather(c + 1)

            pltpu.make_async_copy(
                table.at[idx_v.at[c]], rows_v.at[cm], gsem
            ).wait()

            @pl.when(c >= _NBUF)
            def _():
                pltpu.make_async_copy(
                    trans_v.at[cm],
                    out.at[c - _NBUF, :, pl.ds(base, bw)],
                    wsem,
                ).wait()

            rows = rows_v.at[cm]
            trans = trans_v.at[cm]

            def body(t, cc):
                col = jnp.full((nl,), t, jnp.int32)
                for h in range(dim // nl):
                    v = rows[t, pl.ds(h * nl, nl)]
                    plsc.store_scatter(trans, [iota + h * nl, col], v)
                return cc

            lax.fori_loop(0, bw, body, 0, unroll=8)

            pltpu.async_copy(
                trans_v.at[cm], out.at[c, :, pl.ds(base, bw)], wsem
            )
            return carry

        lax.fori_loop(0, s, loop_body, 0)
        for k in range(_NBUF):
            c = s - _NBUF + k
            pltpu.make_async_copy(
                trans_v.at[c % _NBUF], out.at[c, :, pl.ds(base, bw)], wsem
            ).wait()

    return gather_kernel


def kernel(token_ids, embeddings):
    b, s = token_ids.shape
    v, d = embeddings.shape
    gather_kernel = _build(v, d, b, s)
    out_sdb = gather_kernel(embeddings, token_ids.T)
    return jnp.transpose(out_sdb, (2, 0, 1))
